# baseline (speedup 1.0000x reference)
import functools

import jax
import jax.numpy as jnp
from jax import lax
from jax.experimental import pallas as pl
from jax.experimental.pallas import tpu as pltpu
from jax.experimental.pallas import tpu_sc as plsc

D = 64
SEQ = 200
NC, NS = 2, 16
NW = NC * NS
CHUNK = 800
SUB = 100
NSUB = CHUNK // SUB
LANES = 16
G = D // LANES


@functools.lru_cache(maxsize=None)
def _make_kernel(n_rows):
    n_chunks = n_rows // (NW * CHUNK)
    mesh = plsc.VectorSubcoreMesh(core_axis_name="c", subcore_axis_name="s")

    @functools.partial(
        pl.kernel,
        mesh=mesh,
        out_type=jax.ShapeDtypeStruct((n_rows, D), jnp.float32),
        scratch_types=[
            pltpu.VMEM((NSUB, SUB), jnp.int32),
            pltpu.VMEM((CHUNK, D), jnp.float32),
            pltpu.VMEM((SEQ, D), jnp.float32),
            pltpu.SemaphoreType.DMA,
        ],
        compiler_params=pltpu.CompilerParams(use_tc_tiling_on_sc=False),
    )
    def k(x_hbm, table_hbm, pe_hbm, out_hbm, idx_v, rows_v, pe_v, sem):
        wid = lax.axis_index("s") * NC + lax.axis_index("c")
        pltpu.sync_copy(pe_hbm, pe_v)

        def chunk_body(c, carry):
            gchunk = wid * n_chunks + c
            pltpu.sync_copy(x_hbm.at[gchunk], idx_v)
            cps = [
                pltpu.async_copy(
                    table_hbm.at[idx_v.at[j]],
                    rows_v.at[pl.ds(j * SUB, SUB)],
                    sem,
                )
                for j in range(NSUB)
            ]
            for cp in cps:
                cp.wait()

            def add_body(kk, acc):
                for j in range(G):
                    p = pe_v[kk, pl.ds(LANES * j, LANES)]
                    for s in range(CHUNK // SEQ):
                        r = s * SEQ + kk
                        sl = pl.ds(LANES * j, LANES)
                        rows_v[r, sl] = rows_v[r, sl] + p
                return acc

            lax.fori_loop(0, SEQ, add_body, 0)

            pltpu.sync_copy(rows_v, out_hbm.at[pl.ds(gchunk * CHUNK, CHUNK)])
            return carry

        lax.fori_loop(0, n_chunks, chunk_body, 0)

    return k


def kernel(x, table, pe):
    B, L = x.shape
    n_rows = B * L
    xf = x.reshape(-1).astype(jnp.int32).reshape(n_rows // CHUNK, NSUB, SUB)
    pe2 = pe[0, :L, :]
    out = _make_kernel(n_rows)(xf, table, pe2)
    return out.reshape(B, L, D)


# speedup vs baseline: 1.0003x; 1.0003x over previous
import functools

import jax
import jax.numpy as jnp
from jax import lax
from jax.experimental import pallas as pl
from jax.experimental.pallas import tpu as pltpu
from jax.experimental.pallas import tpu_sc as plsc

D = 64
NC, NS = 2, 16
NW = NC * NS
CSEQ = 4
SUBS = (104, 96)
LANES = 16
G = D // LANES


@functools.lru_cache(maxsize=None)
def _make_kernel(B, L):
    n_chunks = B // (NW * CSEQ)
    mesh = plsc.VectorSubcoreMesh(core_axis_name="c", subcore_axis_name="s")

    @functools.partial(
        pl.kernel,
        mesh=mesh,
        out_type=jax.ShapeDtypeStruct((B, L, D), jnp.float32),
        scratch_types=[
            pltpu.VMEM((CSEQ, L), jnp.int32),
            pltpu.VMEM((CSEQ, L, D), jnp.float32),
            pltpu.VMEM((L, D), jnp.float32),
            pltpu.SemaphoreType.DMA,
        ],
        compiler_params=pltpu.CompilerParams(use_tc_tiling_on_sc=False),
    )
    def k(x_hbm, table_hbm, pe_hbm, out_hbm, idx_v, rows_v, pe_v, sem):
        wid = lax.axis_index("s") * NC + lax.axis_index("c")
        pltpu.sync_copy(pe_hbm.at[0, pl.ds(0, L), :], pe_v)

        def chunk_body(c, carry):
            seq0 = (wid * n_chunks + c) * CSEQ
            pltpu.sync_copy(x_hbm.at[pl.ds(seq0, CSEQ), :], idx_v)
            cps = []
            for s in range(CSEQ):
                o = 0
                for sub in SUBS:
                    cps.append(
                        pltpu.async_copy(
                            table_hbm.at[idx_v.at[s, pl.ds(o, sub)]],
                            rows_v.at[s, pl.ds(o, sub), :],
                            sem,
                        )
                    )
                    o += sub
            for cp in cps:
                cp.wait()

            def add_body(kk, acc):
                for j in range(G):
                    sl = pl.ds(LANES * j, LANES)
                    p = pe_v[kk, sl]
                    for s in range(CSEQ):
                        rows_v[s, kk, sl] = rows_v[s, kk, sl] + p
                return acc

            lax.fori_loop(0, L, add_body, 0)

            pltpu.sync_copy(rows_v, out_hbm.at[pl.ds(seq0, CSEQ)])
            return carry

        lax.fori_loop(0, n_chunks, chunk_body, 0)

    return k


def kernel(x, table, pe):
    B, L = x.shape
    return _make_kernel(B, L)(x.astype(jnp.int32), table, pe)
